# baseline (speedup 1.0000x reference)
import jax
import jax.numpy as jnp
from jax import lax
from jax.experimental import pallas as pl
from jax.experimental.pallas import tpu as pltpu

ROWS = 1024
COLS = 50000
OUT_COLS = 25000
RB = 8
LANES = 128


def _select_body(x_ref, o_ref):
    iota = lax.broadcasted_iota(jnp.int32, (RB, LANES), 1)
    for m in range(OUT_COLS // LANES):
        u0 = m * LANES
        g0 = u0 // 100
        bounds = []
        g = g0
        while 100 * (g + 1) < u0 + LANES:
            bounds.append(100 * (g + 1))
            g += 1
        genes = list(range(g0, g0 + len(bounds) + 1))
        segs = [x_ref[:, u0 + 100 * gg : u0 + 100 * gg + LANES] for gg in genes]
        acc = segs[-1]
        for b, seg in zip(reversed(bounds), reversed(segs[:-1])):
            acc = jnp.where(iota < (b - u0), seg, acc)
        o_ref[:, u0 : u0 + LANES] = acc
    m = OUT_COLS // LANES
    u0 = m * LANES
    nrem = OUT_COLS - u0
    g0 = u0 // 100
    o_ref[:, u0:OUT_COLS] = x_ref[:, u0 + 100 * g0 : u0 + 100 * g0 + nrem]


@jax.jit
def kernel(inputs):
    grid = (ROWS // RB,)
    return pl.pallas_call(
        _select_body,
        grid=grid,
        in_specs=[pl.BlockSpec((RB, COLS), lambda i: (i, 0))],
        out_specs=pl.BlockSpec((RB, OUT_COLS), lambda i: (i, 0)),
        out_shape=jax.ShapeDtypeStruct((ROWS, OUT_COLS), jnp.float32),
        compiler_params=pltpu.CompilerParams(
            dimension_semantics=("arbitrary",),
        ),
    )(inputs)


# speedup vs baseline: 1.1504x; 1.1504x over previous
import jax
import jax.numpy as jnp
from jax import lax
from jax.experimental import pallas as pl
from jax.experimental.pallas import tpu as pltpu

ROWS = 1024
COLS = 50000
OUT_COLS = 25000
RB = 64
LANES = 128


def _select_body(x_ref, o_ref):
    iota = lax.broadcasted_iota(jnp.int32, (RB, LANES), 1)
    for m in range(OUT_COLS // LANES):
        u0 = m * LANES
        g0 = u0 // 100
        bounds = []
        g = g0
        while 100 * (g + 1) < u0 + LANES:
            bounds.append(100 * (g + 1))
            g += 1
        genes = list(range(g0, g0 + len(bounds) + 1))
        segs = [x_ref[:, u0 + 100 * gg : u0 + 100 * gg + LANES] for gg in genes]
        acc = segs[-1]
        for b, seg in zip(reversed(bounds), reversed(segs[:-1])):
            acc = jnp.where(iota < (b - u0), seg, acc)
        o_ref[:, u0 : u0 + LANES] = acc
    m = OUT_COLS // LANES
    u0 = m * LANES
    nrem = OUT_COLS - u0
    g0 = u0 // 100
    o_ref[:, u0:OUT_COLS] = x_ref[:, u0 + 100 * g0 : u0 + 100 * g0 + nrem]


@jax.jit
def kernel(inputs):
    grid = (ROWS // RB,)
    return pl.pallas_call(
        _select_body,
        grid=grid,
        in_specs=[pl.BlockSpec((RB, COLS), lambda i: (i, 0))],
        out_specs=pl.BlockSpec((RB, OUT_COLS), lambda i: (i, 0)),
        out_shape=jax.ShapeDtypeStruct((ROWS, OUT_COLS), jnp.float32),
        compiler_params=pltpu.CompilerParams(
            dimension_semantics=("arbitrary",),
        ),
    )(inputs)
